# Initial kernel scaffold; baseline (speedup 1.0000x reference)
import jax
import jax.numpy as jnp
from jax.experimental import pallas as pl


def kernel(x, aspect_ratio, conv_w, conv_b, pe_ln_w, pe_ln_b, ar_token, t_pos, s_pos, n1_w, n1_b, n2_w, n2_b, loc_qkv_w, loc_proj_w, loc_proj_b, glb_in_w, glb_in_b, glb_out_w, glb_out_b, fc1_w, fc1_b, fc2_w, fc2_b, fn_w, fn_b):
    raise NotImplementedError("write your pallas kernel here")


# speedup vs baseline: 5.0243x; 5.0243x over previous
import functools

import numpy as np
import jax
import jax.numpy as jnp
from jax.experimental import pallas as pl
from jax.experimental.pallas import tpu as pltpu

C = 768
HEADS = 12
DEPTH = 12
HD = C // HEADS
NT = 257
NP = 264
NS = 256
CH = 3351
PK = 1536
SCALE = HD ** -0.5
NEG = -1e30
EPS = 1e-5
CT = 384
MT = -(-CH // CT)

_VMEM_LIMIT = 48 * 1024 * 1024


def _build_consts():
    g = np.arange(16)
    xx = np.repeat(g, 16)
    yy = np.tile(g, 16)
    win = (np.abs(xx[:, None] - xx[None, :]) <= 3) & (np.abs(yy[:, None] - yy[None, :]) <= 3)
    loc_bias = np.where(win, 0.0, NEG).astype(np.float32)
    nx = np.minimum(xx + 3, 15) - np.maximum(xx - 3, 0) + 1
    ny = np.minimum(yy + 3, 15) - np.maximum(yy - 3, 0) + 1
    npad = (49 - nx * ny).astype(np.float32).reshape(NS, 1)
    gb = np.zeros((1, NP), np.float32)
    gb[0, NT:] = NEG
    return loc_bias, npad, gb


_LOC_BIAS_NP, _NPAD_NP, _GLB_BIAS_NP = _build_consts()


def _ln(x, w, b):
    m = jnp.mean(x, axis=-1, keepdims=True)
    xc = x - m
    v = jnp.mean(xc * xc, axis=-1, keepdims=True)
    return xc * jax.lax.rsqrt(v + EPS) * w + b


def _mmt(a, b):
    return jax.lax.dot_general(a, b, (((1,), (1,)), ((), ())),
                               preferred_element_type=jnp.float32)


def _mm(a, b):
    return jax.lax.dot_general(a, b, (((1,), (0,)), ((), ())),
                               preferred_element_type=jnp.float32)


def _gelu(x):
    return 0.5 * x * (1.0 + jax.lax.erf(x * (2.0 ** -0.5)))


def _params(sem):
    return pltpu.CompilerParams(dimension_semantics=sem,
                                vmem_limit_bytes=_VMEM_LIMIT)


def _embed_body(p_ref, w_ref, cb_ref, lnw_ref, lnb_ref, pos_ref, o_ref):
    x = p_ref[...].reshape(2 * NS, PK)
    pe = _mmt(x, w_ref[...]) + cb_ref[...]
    pe = _ln(pe, lnw_ref[...], lnb_ref[...])
    o_ref[...] = pe.reshape(2, NS, C) + pos_ref[...][None]


def _embed(patches, wpe, cb, lnw, lnb, pos):
    full = lambda a: pl.BlockSpec(a.shape, lambda c: (0,) * a.ndim)
    return pl.pallas_call(
        _embed_body,
        grid=(2,),
        in_specs=[pl.BlockSpec((2, NS, PK), lambda c: (c, 0, 0)),
                  full(wpe), full(cb), full(lnw), full(lnb), full(pos)],
        out_specs=pl.BlockSpec((2, NS, C), lambda c: (c, 0, 0)),
        out_shape=jax.ShapeDtypeStruct((4, NS, C), jnp.float32),
        compiler_params=_params(("parallel",)),
        name="d4rt_embed",
    )(patches, wpe, cb, lnw, lnb, pos)


def _gattn_body(h_ref, n1w, n1b, inw, inb, outw, outb, gb_ref, o_ref,
                qkv_s, ao_s):
    x = h_ref[...].reshape(2 * NP, C)
    hn = _ln(x, n1w[...], n1b[...])
    qkv_s[...] = _mmt(hn, inw[...]) + inb[...]
    for b in range(2):
        r0 = b * NP
        for hh in range(HEADS):
            c0 = hh * HD
            q = qkv_s[r0:r0 + NP, c0:c0 + HD]
            k = qkv_s[r0:r0 + NP, C + c0:C + c0 + HD]
            v = qkv_s[r0:r0 + NP, 2 * C + c0:2 * C + c0 + HD]
            s = _mmt(q, k) * SCALE + gb_ref[...]
            m = jnp.max(s, axis=-1, keepdims=True)
            e = jnp.exp(s - m)
            p = e / jnp.sum(e, axis=-1, keepdims=True)
            ao_s[r0:r0 + NP, c0:c0 + HD] = _mm(p, v)
    out = _mmt(ao_s[...], outw[...]) + outb[...]
    o_ref[...] = h_ref[...] + out.reshape(2, NP, C)


def _gattn(h, n1w, n1b, inw, inb, outw, outb, gb):
    full = lambda a: pl.BlockSpec(a.shape, lambda c: (0,) * a.ndim)
    return pl.pallas_call(
        _gattn_body,
        grid=(2,),
        in_specs=[pl.BlockSpec((2, NP, C), lambda c: (c, 0, 0)),
                  full(n1w), full(n1b), full(inw), full(inb),
                  full(outw), full(outb), full(gb)],
        out_specs=pl.BlockSpec((2, NP, C), lambda c: (c, 0, 0)),
        out_shape=jax.ShapeDtypeStruct((4, NP, C), jnp.float32),
        scratch_shapes=[pltpu.VMEM((2 * NP, 3 * C), jnp.float32),
                        pltpu.VMEM((2 * NP, C), jnp.float32)],
        compiler_params=_params(("parallel",)),
        name="d4rt_gattn",
    )(h, n1w, n1b, inw, inb, outw, outb, gb)


def _lattn_body(h_ref, n1w, n1b, qw, pw, pb, lb_ref, npad_ref, o_ref,
                qkv_s, ao_s):
    x = h_ref[:, :NS, :].reshape(2 * NS, C)
    hn = _ln(x, n1w[...], n1b[...])
    qkv_s[...] = _mmt(hn, qw[...])
    for b in range(2):
        r0 = b * NS
        for hh in range(HEADS):
            c0 = hh * HD
            q = qkv_s[r0:r0 + NS, c0:c0 + HD]
            k = qkv_s[r0:r0 + NS, C + c0:C + c0 + HD]
            v = qkv_s[r0:r0 + NS, 2 * C + c0:2 * C + c0 + HD]
            s = _mmt(q, k) * SCALE + lb_ref[...]
            m = jnp.maximum(jnp.max(s, axis=-1, keepdims=True), 0.0)
            e = jnp.exp(s - m)
            den = jnp.sum(e, axis=-1, keepdims=True) + npad_ref[...] * jnp.exp(-m)
            p = e / den
            ao_s[r0:r0 + NS, c0:c0 + HD] = _mm(p, v)
    out = _mmt(ao_s[...], pw[...]) + pb[...]
    o_ref[:, :NS, :] = h_ref[:, :NS, :] + out.reshape(2, NS, C)
    o_ref[:, NS:, :] = h_ref[:, NS:, :]


def _lattn(h, n1w, n1b, qw, pw, pb, lb, npad):
    full = lambda a: pl.BlockSpec(a.shape, lambda c: (0,) * a.ndim)
    return pl.pallas_call(
        _lattn_body,
        grid=(2,),
        in_specs=[pl.BlockSpec((2, NP, C), lambda c: (c, 0, 0)),
                  full(n1w), full(n1b), full(qw), full(pw), full(pb),
                  full(lb), full(npad)],
        out_specs=pl.BlockSpec((2, NP, C), lambda c: (c, 0, 0)),
        out_shape=jax.ShapeDtypeStruct((4, NP, C), jnp.float32),
        scratch_shapes=[pltpu.VMEM((2 * NS, 3 * C), jnp.float32),
                        pltpu.VMEM((2 * NS, C), jnp.float32)],
        compiler_params=_params(("parallel",)),
        name="d4rt_lattn",
    )(h, n1w, n1b, qw, pw, pb, lb, npad)


def _mlp_body(h_ref, n2w, n2b, f1w, f1b, f2w, f2b, fnw, fnb, o_ref, ln_s,
              *, final):
    t = pl.program_id(1)

    @pl.when(t == 0)
    def _():
        ln_s[...] = _ln(h_ref[...].reshape(2 * NP, C), n2w[...], n2b[...])
        o_ref[...] = h_ref[...]

    hcol = _mmt(ln_s[...], f1w[...]) + f1b[...]
    lane = t * CT + jax.lax.broadcasted_iota(jnp.int32, (1, CT), 1)
    valid = lane < CH
    hcol = jnp.where(valid, _gelu(hcol), 0.0)
    f2wv = jnp.where(valid, f2w[...], 0.0)
    part = _mmt(hcol, f2wv)
    acc = o_ref[...] + part.reshape(2, NP, C)

    @pl.when(t != MT - 1)
    def _():
        o_ref[...] = acc

    @pl.when(t == MT - 1)
    def _():
        fin = acc.reshape(2 * NP, C) + f2b[...]
        if final:
            fin = _ln(fin, fnw[...], fnb[...])
        o_ref[...] = fin.reshape(2, NP, C)


def _mlp(h, n2w, n2b, f1w, f1b, f2w, f2b, fnw, fnb, final):
    fix = lambda a: pl.BlockSpec(a.shape, lambda c, t: (0,) * a.ndim)
    return pl.pallas_call(
        functools.partial(_mlp_body, final=final),
        grid=(2, MT),
        in_specs=[pl.BlockSpec((2, NP, C), lambda c, t: (c, 0, 0)),
                  fix(n2w), fix(n2b),
                  pl.BlockSpec((CT, C), lambda c, t: (t, 0)),
                  pl.BlockSpec((1, CT), lambda c, t: (0, t)),
                  pl.BlockSpec((C, CT), lambda c, t: (0, t)),
                  fix(f2b), fix(fnw), fix(fnb)],
        out_specs=pl.BlockSpec((2, NP, C), lambda c, t: (c, 0, 0)),
        out_shape=jax.ShapeDtypeStruct((4, NP, C), jnp.float32),
        scratch_shapes=[pltpu.VMEM((2 * NP, C), jnp.float32)],
        compiler_params=_params(("parallel", "arbitrary")),
        name="d4rt_mlp",
    )(h, n2w, n2b, f1w, f1b, f2w, f2b, fnw, fnb)


def kernel(x, aspect_ratio, conv_w, conv_b, pe_ln_w, pe_ln_b, ar_token,
           t_pos, s_pos, n1_w, n1_b, n2_w, n2_b, loc_qkv_w, loc_proj_w,
           loc_proj_b, glb_in_w, glb_in_b, glb_out_w, glb_out_b,
           fc1_w, fc1_b, fc2_w, fc2_b, fn_w, fn_b):
    B = x.shape[0]
    f32 = jnp.float32

    xp = x.transpose(0, 2, 1, 3, 4).reshape(B, 3, 2, 16, 16, 16, 16)
    patches = xp.transpose(0, 3, 5, 1, 2, 4, 6).reshape(B, NS, PK)
    wpe = conv_w.reshape(C, PK)
    pos = (t_pos[0, 0][None, :] + s_pos[0]).astype(f32)

    row = lambda a: a.reshape(1, -1).astype(f32)
    tokens = _embed(patches, wpe, row(conv_b), row(pe_ln_w), row(pe_ln_b), pos)

    ar_rows = ar_token * (1.0 + aspect_ratio[:, None, None] * 0.1)
    h = jnp.concatenate(
        [tokens, ar_rows.astype(f32), jnp.zeros((B, NP - NT, C), f32)], axis=1)

    lb = jnp.asarray(_LOC_BIAS_NP)
    npad = jnp.asarray(_NPAD_NP)
    gb = jnp.asarray(_GLB_BIAS_NP)

    for i in range(DEPTH):
        if i % 2 == 0:
            e = i // 2
            h = _lattn(h, row(n1_w[i]), row(n1_b[i]), loc_qkv_w[e],
                       loc_proj_w[e], row(loc_proj_b[e]), lb, npad)
        else:
            g = i // 2
            h = _gattn(h, row(n1_w[i]), row(n1_b[i]), glb_in_w[g],
                       row(glb_in_b[g]), glb_out_w[g], row(glb_out_b[g]), gb)
        h = _mlp(h, row(n2_w[i]), row(n2_b[i]), fc1_w[i], row(fc1_b[i]),
                 fc2_w[i], row(fc2_b[i]), row(fn_w), row(fn_b),
                 final=(i == DEPTH - 1))
    return h[:, :NT]


# speedup vs baseline: 6.2772x; 1.2494x over previous
import functools

import numpy as np
import jax
import jax.numpy as jnp
from jax.experimental import pallas as pl
from jax.experimental.pallas import tpu as pltpu

C = 768
HEADS = 12
DEPTH = 12
HD = C // HEADS
NT = 257
NP = 264
NS = 256
CH = 3351
PK = 1536
SCALE = HD ** -0.5
NEG = -1e30
EPS = 1e-5
CT = 384
MT = -(-CH // CT)

_VMEM_LIMIT = 48 * 1024 * 1024


def _build_consts():
    g = np.arange(16)
    xx = np.repeat(g, 16)
    yy = np.tile(g, 16)
    win = (np.abs(xx[:, None] - xx[None, :]) <= 3) & (np.abs(yy[:, None] - yy[None, :]) <= 3)
    loc_bias = np.where(win, 0.0, NEG).astype(np.float32)
    nx = np.minimum(xx + 3, 15) - np.maximum(xx - 3, 0) + 1
    ny = np.minimum(yy + 3, 15) - np.maximum(yy - 3, 0) + 1
    npad = (49 - nx * ny).astype(np.float32).reshape(NS, 1)
    gb = np.zeros((1, NP), np.float32)
    gb[0, NT:] = NEG
    return loc_bias, npad, gb


_LOC_BIAS_NP, _NPAD_NP, _GLB_BIAS_NP = _build_consts()


def _ln(x, w, b):
    m = jnp.mean(x, axis=-1, keepdims=True)
    xc = x - m
    v = jnp.mean(xc * xc, axis=-1, keepdims=True)
    return xc * jax.lax.rsqrt(v + EPS) * w + b


def _mmt(a, b):
    return jax.lax.dot_general(a, b, (((1,), (1,)), ((), ())),
                               preferred_element_type=jnp.float32)


def _mm(a, b):
    return jax.lax.dot_general(a, b, (((1,), (0,)), ((), ())),
                               preferred_element_type=jnp.float32)


def _gelu(x):
    return 0.5 * x * (1.0 + jax.lax.erf(x * (2.0 ** -0.5)))


def _params(sem):
    return pltpu.CompilerParams(dimension_semantics=sem,
                                vmem_limit_bytes=_VMEM_LIMIT)


def _embed_body(p_ref, w_ref, cb_ref, lnw_ref, lnb_ref, pos_ref, o_ref):
    x = p_ref[...].reshape(2 * NS, PK)
    pe = _mmt(x, w_ref[...]) + cb_ref[...]
    pe = _ln(pe, lnw_ref[...], lnb_ref[...])
    o_ref[...] = pe.reshape(2, NS, C) + pos_ref[...][None]


def _embed(patches, wpe, cb, lnw, lnb, pos):
    full = lambda a: pl.BlockSpec(a.shape, lambda c: (0,) * a.ndim)
    return pl.pallas_call(
        _embed_body,
        grid=(2,),
        in_specs=[pl.BlockSpec((2, NS, PK), lambda c: (c, 0, 0)),
                  full(wpe), full(cb), full(lnw), full(lnb), full(pos)],
        out_specs=pl.BlockSpec((2, NS, C), lambda c: (c, 0, 0)),
        out_shape=jax.ShapeDtypeStruct((4, NS, C), jnp.float32),
        compiler_params=_params(("parallel",)),
        name="d4rt_embed",
    )(patches, wpe, cb, lnw, lnb, pos)


def _gattn_body(s_ref, h_ref, n1w, n1b, inw, inb, outw, outb, gb_ref, o_ref,
                qkv_s, ao_s):
    del s_ref
    x = h_ref[...].reshape(2 * NP, C)
    hn = _ln(x, n1w[0], n1b[0])
    qkv_s[...] = _mmt(hn, inw[0]) + inb[0]
    for b in range(2):
        r0 = b * NP
        for hh in range(HEADS):
            c0 = hh * HD
            q = qkv_s[r0:r0 + NP, c0:c0 + HD]
            k = qkv_s[r0:r0 + NP, C + c0:C + c0 + HD]
            v = qkv_s[r0:r0 + NP, 2 * C + c0:2 * C + c0 + HD]
            s = _mmt(q, k) * SCALE + gb_ref[...]
            m = jnp.max(s, axis=-1, keepdims=True)
            e = jnp.exp(s - m)
            p = e / jnp.sum(e, axis=-1, keepdims=True)
            ao_s[r0:r0 + NP, c0:c0 + HD] = _mm(p, v)
    out = _mmt(ao_s[...], outw[0]) + outb[0]
    o_ref[...] = h_ref[...] + out.reshape(2, NP, C)


def _gattn(s, h, n1w3, n1b3, inw, inb3, outw, outb3, gb):
    full = lambda a: pl.BlockSpec(a.shape, lambda c, sr: (0,) * a.ndim)
    return pl.pallas_call(
        _gattn_body,
        grid_spec=pltpu.PrefetchScalarGridSpec(
            num_scalar_prefetch=1,
            grid=(2,),
            in_specs=[pl.BlockSpec((2, NP, C), lambda c, sr: (c, 0, 0)),
                      pl.BlockSpec((1, 1, C), lambda c, sr: (sr[0], 0, 0)),
                      pl.BlockSpec((1, 1, C), lambda c, sr: (sr[0], 0, 0)),
                      pl.BlockSpec((1, 3 * C, C), lambda c, sr: (sr[1], 0, 0)),
                      pl.BlockSpec((1, 1, 3 * C), lambda c, sr: (sr[1], 0, 0)),
                      pl.BlockSpec((1, C, C), lambda c, sr: (sr[1], 0, 0)),
                      pl.BlockSpec((1, 1, C), lambda c, sr: (sr[1], 0, 0)),
                      full(gb)],
            out_specs=pl.BlockSpec((2, NP, C), lambda c, sr: (c, 0, 0)),
            scratch_shapes=[pltpu.VMEM((2 * NP, 3 * C), jnp.float32),
                            pltpu.VMEM((2 * NP, C), jnp.float32)],
        ),
        out_shape=jax.ShapeDtypeStruct((4, NP, C), jnp.float32),
        compiler_params=_params(("parallel",)),
        name="d4rt_gattn",
    )(s, h, n1w3, n1b3, inw, inb3, outw, outb3, gb)


def _lattn_body(s_ref, h_ref, n1w, n1b, qw, pw, pb, lb_ref, npad_ref, o_ref,
                qkv_s, ao_s):
    del s_ref
    x = h_ref[:, :NS, :].reshape(2 * NS, C)
    hn = _ln(x, n1w[0], n1b[0])
    qkv_s[...] = _mmt(hn, qw[0])
    for b in range(2):
        r0 = b * NS
        for hh in range(HEADS):
            c0 = hh * HD
            q = qkv_s[r0:r0 + NS, c0:c0 + HD]
            k = qkv_s[r0:r0 + NS, C + c0:C + c0 + HD]
            v = qkv_s[r0:r0 + NS, 2 * C + c0:2 * C + c0 + HD]
            s = _mmt(q, k) * SCALE + lb_ref[...]
            m = jnp.maximum(jnp.max(s, axis=-1, keepdims=True), 0.0)
            e = jnp.exp(s - m)
            den = jnp.sum(e, axis=-1, keepdims=True) + npad_ref[...] * jnp.exp(-m)
            p = e / den
            ao_s[r0:r0 + NS, c0:c0 + HD] = _mm(p, v)
    out = _mmt(ao_s[...], pw[0]) + pb[0]
    o_ref[:, :NS, :] = h_ref[:, :NS, :] + out.reshape(2, NS, C)
    o_ref[:, NS:, :] = h_ref[:, NS:, :]


def _lattn(s, h, n1w3, n1b3, qw, pw, pb3, lb, npad):
    full = lambda a: pl.BlockSpec(a.shape, lambda c, sr: (0,) * a.ndim)
    return pl.pallas_call(
        _lattn_body,
        grid_spec=pltpu.PrefetchScalarGridSpec(
            num_scalar_prefetch=1,
            grid=(2,),
            in_specs=[pl.BlockSpec((2, NP, C), lambda c, sr: (c, 0, 0)),
                      pl.BlockSpec((1, 1, C), lambda c, sr: (sr[0], 0, 0)),
                      pl.BlockSpec((1, 1, C), lambda c, sr: (sr[0], 0, 0)),
                      pl.BlockSpec((1, 3 * C, C), lambda c, sr: (sr[1], 0, 0)),
                      pl.BlockSpec((1, C, C), lambda c, sr: (sr[1], 0, 0)),
                      pl.BlockSpec((1, 1, C), lambda c, sr: (sr[1], 0, 0)),
                      full(lb), full(npad)],
            out_specs=pl.BlockSpec((2, NP, C), lambda c, sr: (c, 0, 0)),
            scratch_shapes=[pltpu.VMEM((2 * NS, 3 * C), jnp.float32),
                            pltpu.VMEM((2 * NS, C), jnp.float32)],
        ),
        out_shape=jax.ShapeDtypeStruct((4, NP, C), jnp.float32),
        compiler_params=_params(("parallel",)),
        name="d4rt_lattn",
    )(s, h, n1w3, n1b3, qw, pw, pb3, lb, npad)


def _mlp_body(s_ref, h_ref, n2w, n2b, f1w, f1b, f2w, f2b, fnw, fnb, o_ref,
              ln_s, *, final):
    del s_ref
    t = pl.program_id(1)

    @pl.when(t == 0)
    def _():
        ln_s[...] = _ln(h_ref[...].reshape(2 * NP, C), n2w[0], n2b[0])
        o_ref[...] = h_ref[...]

    hcol = _mmt(ln_s[...], f1w[0]) + f1b[0]
    lane = t * CT + jax.lax.broadcasted_iota(jnp.int32, (1, CT), 1)
    valid = lane < CH
    hcol = jnp.where(valid, _gelu(hcol), 0.0)
    f2wv = jnp.where(valid, f2w[0], 0.0)
    part = _mmt(hcol, f2wv)
    acc = o_ref[...] + part.reshape(2, NP, C)

    @pl.when(t != MT - 1)
    def _():
        o_ref[...] = acc

    @pl.when(t == MT - 1)
    def _():
        fin = acc.reshape(2 * NP, C) + f2b[0]
        if final:
            fin = _ln(fin, fnw[...], fnb[...])
        o_ref[...] = fin.reshape(2, NP, C)


def _mlp(s, h, n2w3, n2b3, f1w, f1b3, f2w, f2b3, fnw, fnb, final):
    full = lambda a: pl.BlockSpec(a.shape, lambda c, t, sr: (0,) * a.ndim)
    return pl.pallas_call(
        functools.partial(_mlp_body, final=final),
        grid_spec=pltpu.PrefetchScalarGridSpec(
            num_scalar_prefetch=1,
            grid=(2, MT),
            in_specs=[pl.BlockSpec((2, NP, C), lambda c, t, sr: (c, 0, 0)),
                      pl.BlockSpec((1, 1, C), lambda c, t, sr: (sr[0], 0, 0)),
                      pl.BlockSpec((1, 1, C), lambda c, t, sr: (sr[0], 0, 0)),
                      pl.BlockSpec((1, CT, C), lambda c, t, sr: (sr[0], t, 0)),
                      pl.BlockSpec((1, 1, CT), lambda c, t, sr: (sr[0], 0, t)),
                      pl.BlockSpec((1, C, CT), lambda c, t, sr: (sr[0], 0, t)),
                      pl.BlockSpec((1, 1, C), lambda c, t, sr: (sr[0], 0, 0)),
                      full(fnw), full(fnb)],
            out_specs=pl.BlockSpec((2, NP, C), lambda c, t, sr: (c, 0, 0)),
            scratch_shapes=[pltpu.VMEM((2 * NP, C), jnp.float32)],
        ),
        out_shape=jax.ShapeDtypeStruct((4, NP, C), jnp.float32),
        compiler_params=_params(("parallel", "arbitrary")),
        name="d4rt_mlp",
    )(s, h, n2w3, n2b3, f1w, f1b3, f2w, f2b3, fnw, fnb)


def kernel(x, aspect_ratio, conv_w, conv_b, pe_ln_w, pe_ln_b, ar_token,
           t_pos, s_pos, n1_w, n1_b, n2_w, n2_b, loc_qkv_w, loc_proj_w,
           loc_proj_b, glb_in_w, glb_in_b, glb_out_w, glb_out_b,
           fc1_w, fc1_b, fc2_w, fc2_b, fn_w, fn_b):
    B = x.shape[0]
    f32 = jnp.float32

    xp = x.transpose(0, 2, 1, 3, 4).reshape(B, 3, 2, 16, 16, 16, 16)
    patches = xp.transpose(0, 3, 5, 1, 2, 4, 6).reshape(B, NS, PK)
    wpe = conv_w.reshape(C, PK)
    pos = (t_pos[0, 0][None, :] + s_pos[0]).astype(f32)

    row = lambda a: a.reshape(1, -1).astype(f32)
    tokens = _embed(patches, wpe, row(conv_b), row(pe_ln_w), row(pe_ln_b), pos)

    ar_rows = ar_token * (1.0 + aspect_ratio[:, None, None] * 0.1)
    h = jnp.concatenate(
        [tokens, ar_rows.astype(f32), jnp.zeros((B, NP - NT, C), f32)], axis=1)

    lb = jnp.asarray(_LOC_BIAS_NP)
    npad = jnp.asarray(_NPAD_NP)
    gb = jnp.asarray(_GLB_BIAS_NP)

    n1w3 = n1_w.reshape(DEPTH, 1, C)
    n1b3 = n1_b.reshape(DEPTH, 1, C)
    n2w3 = n2_w.reshape(DEPTH, 1, C)
    n2b3 = n2_b.reshape(DEPTH, 1, C)
    inb3 = glb_in_b.reshape(6, 1, 3 * C)
    outb3 = glb_out_b.reshape(6, 1, C)
    pb3 = loc_proj_b.reshape(6, 1, C)
    f1b3 = fc1_b.reshape(DEPTH, 1, CH)
    f2b3 = fc2_b.reshape(DEPTH, 1, C)
    fnw2 = fn_w.reshape(1, C)
    fnb2 = fn_b.reshape(1, C)

    for i in range(DEPTH):
        s = jnp.array([i, i // 2], jnp.int32)
        if i % 2 == 0:
            h = _lattn(s, h, n1w3, n1b3, loc_qkv_w, loc_proj_w, pb3, lb, npad)
        else:
            h = _gattn(s, h, n1w3, n1b3, glb_in_w, inb3, glb_out_w, outb3, gb)
        h = _mlp(s, h, n2w3, n2b3, fc1_w, f1b3, fc2_w, f2b3, fnw2, fnb2,
                 final=(i == DEPTH - 1))
    return h[:, :NT]
